# baseline (speedup 1.0000x reference)
import jax
import jax.numpy as jnp
from jax import lax
from jax.experimental import pallas as pl
from jax.experimental.pallas import tpu as pltpu
from jax.experimental.pallas import tpu_sc as plsc

_D = 32
_C = 128

_mesh = plsc.VectorSubcoreMesh(core_axis_name="core", subcore_axis_name="subcore")


@jax.jit
def _gather(table, src_t):
  n_tok, n_batch = src_t.shape

  @pl.kernel(
      out_type=jax.ShapeDtypeStruct((n_tok, _D, n_batch), jnp.float32),
      mesh=_mesh,
      scratch_types=[pltpu.VMEM((_C, _D), jnp.float32)],
      compiler_params=pltpu.CompilerParams(
          use_tc_tiling_on_sc=False, needs_layout_passes=False),
  )
  def k(table_hbm, i_hbm, o_hbm, scr):
    def body(i_vmem, o_vmem):
      pltpu.sync_copy(table_hbm.at[i_vmem.at[0]], scr)
      o2d = o_vmem.at[0]
      lo = lax.iota(jnp.int32, 16)
      rots = [(lo + k) % 16 for k in range(16)]

      @plsc.parallel_loop(0, _C, step=16, unroll=2)
      def _(j0):
        for d0 in range(0, _D, 16):
          cols = lo + d0
          for k in range(16):
            r = rots[k] + j0
            vals = plsc.load_gather(scr, [r, cols])
            plsc.store_scatter(o2d, [cols, r], vals)

    pltpu.emit_pipeline(
        body,
        grid=(n_tok, n_batch // _C),
        in_specs=[pl.BlockSpec((1, _C), index_map=lambda t, c: (t, c))],
        out_specs=[pl.BlockSpec((1, _D, _C), index_map=lambda t, c: (t, 0, c))],
        core_axis_name=("core", "subcore"),
        dimension_semantics=(pltpu.PARALLEL, pltpu.PARALLEL),
    )(i_hbm, o_hbm)

  return k(table, src_t)


def kernel(src, table):
  src_t = src.T
  n_tok = src_t.shape[0]
  k_chunks = 4
  step = n_tok // k_chunks
  pieces = [
      _gather(table, src_t[i * step:(i + 1) * step]).transpose(2, 0, 1)
      for i in range(k_chunks)
  ]
  return jnp.concatenate(pieces, axis=1)


# speedup vs baseline: 1.0136x; 1.0136x over previous
import jax
import jax.numpy as jnp
from jax import lax
from jax.experimental import pallas as pl
from jax.experimental.pallas import tpu as pltpu
from jax.experimental.pallas import tpu_sc as plsc

_D = 32
_C = 128

_mesh = plsc.VectorSubcoreMesh(core_axis_name="core", subcore_axis_name="subcore")


@jax.jit
def _gather(table, src_t):
  n_tok, n_batch = src_t.shape

  @pl.kernel(
      out_type=jax.ShapeDtypeStruct((n_tok, _D, n_batch), jnp.float32),
      mesh=_mesh,
      scratch_types=[pltpu.VMEM((_C, _D), jnp.float32)],
      compiler_params=pltpu.CompilerParams(
          use_tc_tiling_on_sc=False, needs_layout_passes=False),
  )
  def k(table_hbm, i_hbm, o_hbm, scr):
    def body(i_vmem, o_vmem):
      pltpu.sync_copy(table_hbm.at[i_vmem.at[0]], scr)
      o2d = o_vmem.at[0]
      lo = lax.iota(jnp.int32, 16)
      rots = [(lo + k) % 16 for k in range(16)]

      @plsc.parallel_loop(0, _C, step=16, unroll=2)
      def _(j0):
        for d0 in range(0, _D, 16):
          cols = lo + d0
          for k in range(16):
            r = rots[k] + j0
            vals = plsc.load_gather(scr, [r, cols])
            plsc.store_scatter(o2d, [cols, r], vals)

    pltpu.emit_pipeline(
        body,
        grid=(n_tok, n_batch // _C),
        in_specs=[pl.BlockSpec((1, _C), index_map=lambda t, c: (t, c))],
        out_specs=[pl.BlockSpec((1, _D, _C), index_map=lambda t, c: (t, 0, c))],
        core_axis_name=("core", "subcore"),
        dimension_semantics=(pltpu.PARALLEL, pltpu.PARALLEL),
    )(i_hbm, o_hbm)

  return k(table, src_t)


def kernel(src, table):
  out_t = _gather(table, src.T)
  return out_t.transpose(2, 0, 1)
